# baseline (speedup 1.0000x reference)
import functools

import jax
import jax.numpy as jnp
from jax import lax
from jax.experimental import pallas as pl
from jax.experimental.pallas import tpu as pltpu
from jax.experimental.pallas import tpu_sc as plsc

N_NODES = 10000
N_EDGES = 320000
HIDDEN = 128
NUM_LAYERS = 3
NUM_GRAPHS = 128
OUT_DIM = 10
N_ATOM_FEATS = 9
N_BOND_FEATS = 3
ATOM_VOCAB = 128
BOND_VOCAB = 16

NC = 2
NS = 16
NW = NC * NS
CHUNK = 80
CPW = 126
CPW0 = 106
CPW1 = 146
E_PAD = NW * CPW * CHUNK
N_PAD = 10240
ROWS_PER_TILE = N_PAD // NS
VEC = 16


def _atom_encoder_body(x_ref, emb_ref, out_ref):
    xb = x_ref[...]
    blk = xb.shape[0]
    acc = jnp.zeros((blk, HIDDEN), dtype=jnp.float32)
    iota = lax.broadcasted_iota(jnp.int32, (blk, ATOM_VOCAB), 1)
    for f in range(N_ATOM_FEATS):
        onehot = (xb[:, f][:, None] == iota).astype(jnp.float32)
        acc = acc + jnp.dot(onehot, emb_ref[f],
                            preferred_element_type=jnp.float32)
    out_ref[...] = acc


def _atom_encoder(x, atom_emb):
    blk = 1000
    grid = N_NODES // blk
    return pl.pallas_call(
        _atom_encoder_body,
        grid=(grid,),
        in_specs=[
            pl.BlockSpec((blk, N_ATOM_FEATS), lambda i: (i, 0)),
            pl.BlockSpec((N_ATOM_FEATS, ATOM_VOCAB, HIDDEN), lambda i: (0, 0, 0)),
        ],
        out_specs=pl.BlockSpec((blk, HIDDEN), lambda i: (i, 0)),
        out_shape=jax.ShapeDtypeStruct((N_NODES, HIDDEN), jnp.float32),
    )(x, atom_emb)


def _bond_tables_body(bond_ref, out_ref):
    for l in range(NUM_LAYERS):
        t0 = bond_ref[l, 0]
        t1 = bond_ref[l, 1]
        t2 = bond_ref[l, 2]
        t = (t2[:, None, None, :] + t1[None, :, None, :] + t0[None, None, :, :])
        out_ref[l] = t.reshape(BOND_VOCAB ** 3, HIDDEN)


def _bond_tables(bond_emb):
    return pl.pallas_call(
        _bond_tables_body,
        out_shape=jax.ShapeDtypeStruct(
            (NUM_LAYERS, BOND_VOCAB ** 3, HIDDEN), jnp.float32),
    )(bond_emb)


def _sc_edge_body(h_hbm, src_hbm, dst_hbm, eidx_hbm, table_hbm,
                  out0_hbm, out1_hbm,
                  src_v, dst_v, eidx_v, hrows_v, erows_v,
                  sem_i, sem_g, sem_s,
                  aggr_sh):
    c = lax.axis_index("c")
    s = lax.axis_index("s")
    chunk0 = lax.select(c == 0, s * CPW0, NS * CPW0 + s * CPW1)
    my_rounds = lax.select(c == 0, CPW0 // 2, CPW1 // 2)

    @plsc.parallel_loop(0, CHUNK, 1, unroll=4)
    def _zfill(j):
        for k in range(HIDDEN // VEC):
            hrows_v[0, j, pl.ds(k * VEC, VEC)] = jnp.zeros((VEC,), jnp.float32)
    for r in range(ROWS_PER_TILE // CHUNK):
        pltpu.sync_copy(
            hrows_v.at[0], aggr_sh.at[pl.ds(s * ROWS_PER_TILE + r * CHUNK, CHUNK)])
    plsc.subcore_barrier()

    def load_idx(b, ci):
        base = (chunk0 + ci) * CHUNK
        c0 = pltpu.async_copy(src_hbm.at[pl.ds(base, CHUNK)], src_v.at[b], sem_i)
        c1 = pltpu.async_copy(eidx_hbm.at[pl.ds(base, CHUNK)], eidx_v.at[b], sem_i)
        c2 = pltpu.async_copy(dst_hbm.at[pl.ds(base, CHUNK)], dst_v.at[b], sem_i)
        c0.wait(); c1.wait(); c2.wait()

    def fire_gather(b):
        pltpu.async_copy(h_hbm.at[src_v.at[b]], hrows_v.at[b], sem_g)
        pltpu.async_copy(table_hbm.at[eidx_v.at[b]], erows_v.at[b], sem_g)

    def wait_gather(b):
        pltpu.make_async_copy(h_hbm.at[src_v.at[b]], hrows_v.at[b], sem_g).wait()
        pltpu.make_async_copy(table_hbm.at[eidx_v.at[b]], erows_v.at[b], sem_g).wait()

    def fire_scatter(b):
        pltpu.async_copy(hrows_v.at[b], aggr_sh.at[dst_v.at[b]], sem_s, add=True)

    def wait_scatter(b):
        pltpu.make_async_copy(hrows_v.at[b], aggr_sh.at[dst_v.at[b]], sem_s).wait()

    def compute(b):
        @plsc.parallel_loop(0, CHUNK, 1, unroll=4)
        def _(j):
            for k in range(HIDDEN // VEC):
                sl = pl.ds(k * VEC, VEC)
                hrows_v[b, j, sl] = jnp.maximum(
                    hrows_v[b, j, sl] + erows_v[b, j, sl], 0.0)

    load_idx(0, 0)
    fire_gather(0)

    def round_body(r, _):
        @pl.when(r >= 1)
        def _():
            wait_scatter(1)
        load_idx(1, 2 * r + 1)
        fire_gather(1)
        wait_gather(0)
        compute(0)
        fire_scatter(0)

        wait_scatter(0)
        @pl.when(r <= my_rounds - 2)
        def _():
            load_idx(0, 2 * r + 2)
            fire_gather(0)
        wait_gather(1)
        compute(1)
        fire_scatter(1)
        return 0
    lax.fori_loop(0, my_rounds, round_body, 0)
    wait_scatter(1)
    plsc.subcore_barrier()

    @pl.when(c == 0)
    def _():
        pltpu.sync_copy(aggr_sh.at[pl.ds(s * ROWS_PER_TILE, ROWS_PER_TILE)],
                        out0_hbm.at[pl.ds(s * ROWS_PER_TILE, ROWS_PER_TILE)])

    @pl.when(c == 1)
    def _():
        pltpu.sync_copy(aggr_sh.at[pl.ds(s * ROWS_PER_TILE, ROWS_PER_TILE)],
                        out1_hbm.at[pl.ds(s * ROWS_PER_TILE, ROWS_PER_TILE)])


@functools.cache
def _get_sc_edge():
  return pl.kernel(
    _sc_edge_body,
    out_type=(
        jax.ShapeDtypeStruct((N_PAD, HIDDEN), jnp.float32),
        jax.ShapeDtypeStruct((N_PAD, HIDDEN), jnp.float32),
    ),
    mesh=plsc.VectorSubcoreMesh(core_axis_name="c", subcore_axis_name="s",
                                num_cores=NC, num_subcores=NS),
    scratch_types=[
        pltpu.VMEM((2, CHUNK), jnp.int32),
        pltpu.VMEM((2, CHUNK), jnp.int32),
        pltpu.VMEM((2, CHUNK), jnp.int32),
        pltpu.VMEM((2, CHUNK, HIDDEN), jnp.float32),
        pltpu.VMEM((2, CHUNK, HIDDEN), jnp.float32),
        pltpu.SemaphoreType.DMA,
        pltpu.SemaphoreType.DMA,
        pltpu.SemaphoreType.DMA,
        pltpu.VMEM_SHARED((N_PAD, HIDDEN), jnp.float32),
    ],
  )


def _mlp_body(h_ref, a0_ref, a1_ref, scale_ref, w1_ref, b1_ref, w2_ref, b2_ref,
              out_ref):
    z = h_ref[...] * scale_ref[0, 0] + a0_ref[...] + a1_ref[...]
    t = jnp.dot(z, w1_ref[...], preferred_element_type=jnp.float32) + b1_ref[...]
    t = jnp.maximum(t, 0.0)
    out_ref[...] = (jnp.dot(t, w2_ref[...], preferred_element_type=jnp.float32)
                    + b2_ref[...])


def _mlp(h, a0, a1, scale, w1, b1, w2, b2):
    blk = 1000
    grid = N_NODES // blk
    return pl.pallas_call(
        _mlp_body,
        grid=(grid,),
        in_specs=[
            pl.BlockSpec((blk, HIDDEN), lambda i: (i, 0)),
            pl.BlockSpec((blk, HIDDEN), lambda i: (i, 0)),
            pl.BlockSpec((blk, HIDDEN), lambda i: (i, 0)),
            pl.BlockSpec((1, 1), lambda i: (0, 0)),
            pl.BlockSpec((HIDDEN, 2 * HIDDEN), lambda i: (0, 0)),
            pl.BlockSpec((1, 2 * HIDDEN), lambda i: (0, 0)),
            pl.BlockSpec((2 * HIDDEN, HIDDEN), lambda i: (0, 0)),
            pl.BlockSpec((1, HIDDEN), lambda i: (0, 0)),
        ],
        out_specs=pl.BlockSpec((blk, HIDDEN), lambda i: (i, 0)),
        out_shape=jax.ShapeDtypeStruct((N_NODES, HIDDEN), jnp.float32),
    )(h, a0, a1, scale, w1, b1, w2, b2)


def _readout_body(h_ref, batch_ref, wc_ref, bc_ref, logits_ref, gf_ref):
    onehot = (batch_ref[...] ==
              lax.broadcasted_iota(jnp.int32, (N_NODES, NUM_GRAPHS), 1)
              ).astype(jnp.float32)
    sums = lax.dot_general(onehot, h_ref[...], (((0,), (0,)), ((), ())),
                           preferred_element_type=jnp.float32)
    counts = jnp.sum(onehot, axis=0)[:, None]
    gf = sums / jnp.maximum(counts, 1.0)
    logits_ref[...] = (jnp.dot(gf, wc_ref[...], preferred_element_type=jnp.float32)
                       + bc_ref[...])
    gf_ref[...] = gf


def _readout(h, batch2d, wc, bc):
    return pl.pallas_call(
        _readout_body,
        out_shape=(
            jax.ShapeDtypeStruct((NUM_GRAPHS, OUT_DIM), jnp.float32),
            jax.ShapeDtypeStruct((NUM_GRAPHS, HIDDEN), jnp.float32),
        ),
    )(h, batch2d, wc, bc)


def kernel(x, edge_index, edge_attr, batch, atom_emb, bond_emb, eps,
           W1, b1, W2, b2, Wc, bc):
    src = edge_index[0]
    dst = edge_index[1]
    eidx = (edge_attr[:, 0] + BOND_VOCAB * edge_attr[:, 1]
            + BOND_VOCAB * BOND_VOCAB * edge_attr[:, 2])
    pad = E_PAD - N_EDGES
    src_p = jnp.concatenate([src, jnp.zeros((pad,), jnp.int32)])
    dst_p = jnp.concatenate([dst, jnp.full((pad,), N_NODES, jnp.int32)])
    eidx_p = jnp.concatenate([eidx, jnp.zeros((pad,), jnp.int32)])

    tables = _bond_tables(bond_emb)
    h = _atom_encoder(x, atom_emb)

    b1_2d = b1.reshape(NUM_LAYERS, 1, 2 * HIDDEN)
    b2_2d = b2.reshape(NUM_LAYERS, 1, HIDDEN)
    scales = (1.0 + eps).reshape(NUM_LAYERS, 1, 1)

    for l in range(NUM_LAYERS):
        a0, a1 = _get_sc_edge()(h, src_p, dst_p, eidx_p, tables[l])
        h = _mlp(h, a0, a1, scales[l], W1[l], b1_2d[l], W2[l], b2_2d[l])

    logits, gf = _readout(h, batch[:, None], Wc, bc)
    return (logits, gf)


# speedup vs baseline: 1.1407x; 1.1407x over previous
import functools

import jax
import jax.numpy as jnp
from jax import lax
from jax.experimental import pallas as pl
from jax.experimental.pallas import tpu as pltpu
from jax.experimental.pallas import tpu_sc as plsc

N_NODES = 10000
N_EDGES = 320000
HIDDEN = 128
NUM_LAYERS = 3
NUM_GRAPHS = 128
OUT_DIM = 10
N_ATOM_FEATS = 9
N_BOND_FEATS = 3
ATOM_VOCAB = 128
BOND_VOCAB = 16

NC = 2
NS = 16
NW = NC * NS
CHUNK = 80
CPW = 126
CPW0 = 146
CPW1 = 106
E_PAD = NW * CPW * CHUNK
N_PAD = 10240
ROWS_PER_TILE = N_PAD // NS
VEC = 16


def _atom_encoder_body(x_ref, emb_ref, out_ref):
    xb = x_ref[...]
    blk = xb.shape[0]
    acc = jnp.zeros((blk, HIDDEN), dtype=jnp.float32)
    iota = lax.broadcasted_iota(jnp.int32, (blk, ATOM_VOCAB), 1)
    for f in range(N_ATOM_FEATS):
        onehot = (xb[:, f][:, None] == iota).astype(jnp.float32)
        acc = acc + jnp.dot(onehot, emb_ref[f],
                            preferred_element_type=jnp.float32)
    out_ref[...] = acc


def _atom_encoder(x, atom_emb):
    blk = 1000
    grid = N_NODES // blk
    return pl.pallas_call(
        _atom_encoder_body,
        grid=(grid,),
        in_specs=[
            pl.BlockSpec((blk, N_ATOM_FEATS), lambda i: (i, 0)),
            pl.BlockSpec((N_ATOM_FEATS, ATOM_VOCAB, HIDDEN), lambda i: (0, 0, 0)),
        ],
        out_specs=pl.BlockSpec((blk, HIDDEN), lambda i: (i, 0)),
        out_shape=jax.ShapeDtypeStruct((N_NODES, HIDDEN), jnp.float32),
    )(x, atom_emb)


def _bond_tables_body(bond_ref, out_ref):
    for l in range(NUM_LAYERS):
        t0 = bond_ref[l, 0]
        t1 = bond_ref[l, 1]
        t2 = bond_ref[l, 2]
        t = (t2[:, None, None, :] + t1[None, :, None, :] + t0[None, None, :, :])
        out_ref[l] = t.reshape(BOND_VOCAB ** 3, HIDDEN)


def _bond_tables(bond_emb):
    return pl.pallas_call(
        _bond_tables_body,
        out_shape=jax.ShapeDtypeStruct(
            (NUM_LAYERS, BOND_VOCAB ** 3, HIDDEN), jnp.float32),
    )(bond_emb)


def _sc_edge_body(h_hbm, src_hbm, dst_hbm, eidx_hbm, table_hbm,
                  out0_hbm, out1_hbm,
                  src_v, dst_v, eidx_v, hrows_v, erows_v,
                  sem_i, sem_g, sem_s,
                  aggr_sh):
    c = lax.axis_index("c")
    s = lax.axis_index("s")
    chunk0 = lax.select(c == 0, s * CPW0, NS * CPW0 + s * CPW1)
    my_rounds = lax.select(c == 0, CPW0 // 2, CPW1 // 2)

    @plsc.parallel_loop(0, CHUNK, 1, unroll=4)
    def _zfill(j):
        for k in range(HIDDEN // VEC):
            hrows_v[0, j, pl.ds(k * VEC, VEC)] = jnp.zeros((VEC,), jnp.float32)
    for r in range(ROWS_PER_TILE // CHUNK):
        pltpu.sync_copy(
            hrows_v.at[0], aggr_sh.at[pl.ds(s * ROWS_PER_TILE + r * CHUNK, CHUNK)])
    plsc.subcore_barrier()

    def load_idx(b, ci):
        base = (chunk0 + ci) * CHUNK
        c0 = pltpu.async_copy(src_hbm.at[pl.ds(base, CHUNK)], src_v.at[b], sem_i)
        c1 = pltpu.async_copy(eidx_hbm.at[pl.ds(base, CHUNK)], eidx_v.at[b], sem_i)
        c2 = pltpu.async_copy(dst_hbm.at[pl.ds(base, CHUNK)], dst_v.at[b], sem_i)
        c0.wait(); c1.wait(); c2.wait()

    def fire_gather(b):
        pltpu.async_copy(h_hbm.at[src_v.at[b]], hrows_v.at[b], sem_g)
        pltpu.async_copy(table_hbm.at[eidx_v.at[b]], erows_v.at[b], sem_g)

    def wait_gather(b):
        pltpu.make_async_copy(h_hbm.at[src_v.at[b]], hrows_v.at[b], sem_g).wait()
        pltpu.make_async_copy(table_hbm.at[eidx_v.at[b]], erows_v.at[b], sem_g).wait()

    def fire_scatter(b):
        pltpu.async_copy(hrows_v.at[b], aggr_sh.at[dst_v.at[b]], sem_s, add=True)

    def wait_scatter(b):
        pltpu.make_async_copy(hrows_v.at[b], aggr_sh.at[dst_v.at[b]], sem_s).wait()

    def compute(b):
        @plsc.parallel_loop(0, CHUNK, 1, unroll=4)
        def _(j):
            for k in range(HIDDEN // VEC):
                sl = pl.ds(k * VEC, VEC)
                hrows_v[b, j, sl] = jnp.maximum(
                    hrows_v[b, j, sl] + erows_v[b, j, sl], 0.0)

    load_idx(0, 0)
    fire_gather(0)

    def round_body(r, _):
        @pl.when(r >= 1)
        def _():
            wait_scatter(1)
        load_idx(1, 2 * r + 1)
        fire_gather(1)
        wait_gather(0)
        compute(0)
        fire_scatter(0)

        wait_scatter(0)
        @pl.when(r <= my_rounds - 2)
        def _():
            load_idx(0, 2 * r + 2)
            fire_gather(0)
        wait_gather(1)
        compute(1)
        fire_scatter(1)
        return 0
    lax.fori_loop(0, my_rounds, round_body, 0)
    wait_scatter(1)
    plsc.subcore_barrier()

    @pl.when(c == 0)
    def _():
        pltpu.sync_copy(aggr_sh.at[pl.ds(s * ROWS_PER_TILE, ROWS_PER_TILE)],
                        out0_hbm.at[pl.ds(s * ROWS_PER_TILE, ROWS_PER_TILE)])

    @pl.when(c == 1)
    def _():
        pltpu.sync_copy(aggr_sh.at[pl.ds(s * ROWS_PER_TILE, ROWS_PER_TILE)],
                        out1_hbm.at[pl.ds(s * ROWS_PER_TILE, ROWS_PER_TILE)])


@functools.cache
def _get_sc_edge():
  return pl.kernel(
    _sc_edge_body,
    out_type=(
        jax.ShapeDtypeStruct((N_PAD, HIDDEN), jnp.float32),
        jax.ShapeDtypeStruct((N_PAD, HIDDEN), jnp.float32),
    ),
    mesh=plsc.VectorSubcoreMesh(core_axis_name="c", subcore_axis_name="s",
                                num_cores=NC, num_subcores=NS),
    scratch_types=[
        pltpu.VMEM((2, CHUNK), jnp.int32),
        pltpu.VMEM((2, CHUNK), jnp.int32),
        pltpu.VMEM((2, CHUNK), jnp.int32),
        pltpu.VMEM((2, CHUNK, HIDDEN), jnp.float32),
        pltpu.VMEM((2, CHUNK, HIDDEN), jnp.float32),
        pltpu.SemaphoreType.DMA,
        pltpu.SemaphoreType.DMA,
        pltpu.SemaphoreType.DMA,
        pltpu.VMEM_SHARED((N_PAD, HIDDEN), jnp.float32),
    ],
  )


def _mlp_body(h_ref, a0_ref, a1_ref, scale_ref, w1_ref, b1_ref, w2_ref, b2_ref,
              out_ref):
    z = h_ref[...] * scale_ref[0, 0] + a0_ref[...] + a1_ref[...]
    t = jnp.dot(z, w1_ref[...], preferred_element_type=jnp.float32) + b1_ref[...]
    t = jnp.maximum(t, 0.0)
    out_ref[...] = (jnp.dot(t, w2_ref[...], preferred_element_type=jnp.float32)
                    + b2_ref[...])


def _mlp(h, a0, a1, scale, w1, b1, w2, b2):
    blk = 1000
    grid = N_NODES // blk
    return pl.pallas_call(
        _mlp_body,
        grid=(grid,),
        in_specs=[
            pl.BlockSpec((blk, HIDDEN), lambda i: (i, 0)),
            pl.BlockSpec((blk, HIDDEN), lambda i: (i, 0)),
            pl.BlockSpec((blk, HIDDEN), lambda i: (i, 0)),
            pl.BlockSpec((1, 1), lambda i: (0, 0)),
            pl.BlockSpec((HIDDEN, 2 * HIDDEN), lambda i: (0, 0)),
            pl.BlockSpec((1, 2 * HIDDEN), lambda i: (0, 0)),
            pl.BlockSpec((2 * HIDDEN, HIDDEN), lambda i: (0, 0)),
            pl.BlockSpec((1, HIDDEN), lambda i: (0, 0)),
        ],
        out_specs=pl.BlockSpec((blk, HIDDEN), lambda i: (i, 0)),
        out_shape=jax.ShapeDtypeStruct((N_NODES, HIDDEN), jnp.float32),
    )(h, a0, a1, scale, w1, b1, w2, b2)


def _readout_body(h_ref, batch_ref, wc_ref, bc_ref, logits_ref, gf_ref):
    onehot = (batch_ref[...] ==
              lax.broadcasted_iota(jnp.int32, (N_NODES, NUM_GRAPHS), 1)
              ).astype(jnp.float32)
    sums = lax.dot_general(onehot, h_ref[...], (((0,), (0,)), ((), ())),
                           preferred_element_type=jnp.float32)
    counts = jnp.sum(onehot, axis=0)[:, None]
    gf = sums / jnp.maximum(counts, 1.0)
    logits_ref[...] = (jnp.dot(gf, wc_ref[...], preferred_element_type=jnp.float32)
                       + bc_ref[...])
    gf_ref[...] = gf


def _readout(h, batch2d, wc, bc):
    return pl.pallas_call(
        _readout_body,
        out_shape=(
            jax.ShapeDtypeStruct((NUM_GRAPHS, OUT_DIM), jnp.float32),
            jax.ShapeDtypeStruct((NUM_GRAPHS, HIDDEN), jnp.float32),
        ),
    )(h, batch2d, wc, bc)


def kernel(x, edge_index, edge_attr, batch, atom_emb, bond_emb, eps,
           W1, b1, W2, b2, Wc, bc):
    src = edge_index[0]
    dst = edge_index[1]
    eidx = (edge_attr[:, 0] + BOND_VOCAB * edge_attr[:, 1]
            + BOND_VOCAB * BOND_VOCAB * edge_attr[:, 2])
    pad = E_PAD - N_EDGES
    src_p = jnp.concatenate([src, jnp.zeros((pad,), jnp.int32)])
    dst_p = jnp.concatenate([dst, jnp.full((pad,), N_NODES, jnp.int32)])
    eidx_p = jnp.concatenate([eidx, jnp.zeros((pad,), jnp.int32)])

    tables = _bond_tables(bond_emb)
    h = _atom_encoder(x, atom_emb)

    b1_2d = b1.reshape(NUM_LAYERS, 1, 2 * HIDDEN)
    b2_2d = b2.reshape(NUM_LAYERS, 1, HIDDEN)
    scales = (1.0 + eps).reshape(NUM_LAYERS, 1, 1)

    for l in range(NUM_LAYERS):
        a0, a1 = _get_sc_edge()(h, src_p, dst_p, eidx_p, tables[l])
        h = _mlp(h, a0, a1, scales[l], W1[l], b1_2d[l], W2[l], b2_2d[l])

    logits, gf = _readout(h, batch[:, None], Wc, bc)
    return (logits, gf)


# speedup vs baseline: 1.1601x; 1.0171x over previous
import functools

import jax
import jax.numpy as jnp
from jax import lax
from jax.experimental import pallas as pl
from jax.experimental.pallas import tpu as pltpu
from jax.experimental.pallas import tpu_sc as plsc

N_NODES = 10000
N_EDGES = 320000
HIDDEN = 128
NUM_LAYERS = 3
NUM_GRAPHS = 128
OUT_DIM = 10
N_ATOM_FEATS = 9
N_BOND_FEATS = 3
ATOM_VOCAB = 128
BOND_VOCAB = 16

NC = 2
NS = 16
NW = NC * NS
CHUNK = 80
CPW = 126
CPW0 = 148
CPW1 = 104
E_PAD = NW * CPW * CHUNK
N_PAD = 10240
ROWS_PER_TILE = N_PAD // NS
VEC = 16


def _atom_encoder_body(x_ref, emb_ref, out_ref):
    xb = x_ref[...]
    blk = xb.shape[0]
    acc = jnp.zeros((blk, HIDDEN), dtype=jnp.float32)
    iota = lax.broadcasted_iota(jnp.int32, (blk, ATOM_VOCAB), 1)
    for f in range(N_ATOM_FEATS):
        onehot = (xb[:, f][:, None] == iota).astype(jnp.float32)
        acc = acc + jnp.dot(onehot, emb_ref[f],
                            preferred_element_type=jnp.float32)
    out_ref[...] = acc


def _atom_encoder(x, atom_emb):
    blk = 1000
    grid = N_NODES // blk
    return pl.pallas_call(
        _atom_encoder_body,
        grid=(grid,),
        in_specs=[
            pl.BlockSpec((blk, N_ATOM_FEATS), lambda i: (i, 0)),
            pl.BlockSpec((N_ATOM_FEATS, ATOM_VOCAB, HIDDEN), lambda i: (0, 0, 0)),
        ],
        out_specs=pl.BlockSpec((blk, HIDDEN), lambda i: (i, 0)),
        out_shape=jax.ShapeDtypeStruct((N_NODES, HIDDEN), jnp.float32),
    )(x, atom_emb)


def _bond_tables_body(bond_ref, out_ref):
    for l in range(NUM_LAYERS):
        t0 = bond_ref[l, 0]
        t1 = bond_ref[l, 1]
        t2 = bond_ref[l, 2]
        t = (t2[:, None, None, :] + t1[None, :, None, :] + t0[None, None, :, :])
        out_ref[l] = t.reshape(BOND_VOCAB ** 3, HIDDEN)


def _bond_tables(bond_emb):
    return pl.pallas_call(
        _bond_tables_body,
        out_shape=jax.ShapeDtypeStruct(
            (NUM_LAYERS, BOND_VOCAB ** 3, HIDDEN), jnp.float32),
    )(bond_emb)


def _sc_edge_body(h_hbm, src_hbm, dst_hbm, eidx_hbm, table_hbm,
                  out0_hbm, out1_hbm,
                  src_v, dst_v, eidx_v, hrows_v, erows_v,
                  sem_i, sem_g, sem_s,
                  aggr_sh):
    c = lax.axis_index("c")
    s = lax.axis_index("s")
    chunk0 = lax.select(c == 0, s * CPW0, NS * CPW0 + s * CPW1)
    my_rounds = lax.select(c == 0, CPW0 // 4, CPW1 // 4)

    @plsc.parallel_loop(0, CHUNK, 1, unroll=4)
    def _zfill(j):
        for k in range(HIDDEN // VEC):
            hrows_v[0, j, pl.ds(k * VEC, VEC)] = jnp.zeros((VEC,), jnp.float32)
    for r in range(ROWS_PER_TILE // CHUNK):
        pltpu.sync_copy(
            hrows_v.at[0], aggr_sh.at[pl.ds(s * ROWS_PER_TILE + r * CHUNK, CHUNK)])
    plsc.subcore_barrier()

    def fire_idx(q, ci):
        base = (chunk0 + ci) * CHUNK
        pltpu.async_copy(src_hbm.at[pl.ds(base, CHUNK)], src_v.at[q], sem_i)
        pltpu.async_copy(eidx_hbm.at[pl.ds(base, CHUNK)], eidx_v.at[q], sem_i)
        pltpu.async_copy(dst_hbm.at[pl.ds(base, CHUNK)], dst_v.at[q], sem_i)

    def wait_idx(q, ci):
        base = (chunk0 + ci) * CHUNK
        pltpu.make_async_copy(src_hbm.at[pl.ds(base, CHUNK)], src_v.at[q], sem_i).wait()
        pltpu.make_async_copy(eidx_hbm.at[pl.ds(base, CHUNK)], eidx_v.at[q], sem_i).wait()
        pltpu.make_async_copy(dst_hbm.at[pl.ds(base, CHUNK)], dst_v.at[q], sem_i).wait()

    def fire_gather(b, q):
        pltpu.async_copy(h_hbm.at[src_v.at[q]], hrows_v.at[b], sem_g)
        pltpu.async_copy(table_hbm.at[eidx_v.at[q]], erows_v.at[b], sem_g)

    def wait_gather(b, q):
        pltpu.make_async_copy(h_hbm.at[src_v.at[q]], hrows_v.at[b], sem_g).wait()
        pltpu.make_async_copy(table_hbm.at[eidx_v.at[q]], erows_v.at[b], sem_g).wait()

    def fire_scatter(b, q):
        pltpu.async_copy(hrows_v.at[b], aggr_sh.at[dst_v.at[q]], sem_s, add=True)

    def wait_scatter(b, q):
        pltpu.make_async_copy(hrows_v.at[b], aggr_sh.at[dst_v.at[q]], sem_s).wait()

    def compute(b):
        @plsc.parallel_loop(0, CHUNK, 1, unroll=4)
        def _(j):
            for k in range(HIDDEN // VEC):
                sl = pl.ds(k * VEC, VEC)
                hrows_v[b, j, sl] = jnp.maximum(
                    hrows_v[b, j, sl] + erows_v[b, j, sl], 0.0)

    fire_idx(0, 0)
    wait_idx(0, 0)
    fire_gather(0, 0)
    fire_idx(1, 1)

    def round_body(r, _):
        for j in range(4):
            ci = 4 * r + j
            b, q = j % 2, j
            bn, qn = (j + 1) % 2, (j + 1) % 4
            bp, qp = (j - 1) % 2, (j - 1) % 4

            if j == 0:
                @pl.when(r >= 1)
                def _():
                    wait_scatter(bp, qp)
            else:
                wait_scatter(bp, qp)

            if j <= 2:
                wait_idx(qn, ci + 1)
                fire_gather(bn, qn)
            else:
                @pl.when(r <= my_rounds - 2)
                def _():
                    wait_idx(qn, ci + 1)
                    fire_gather(bn, qn)

            wait_gather(b, q)
            compute(b)
            fire_scatter(b, q)

            if j <= 1:
                fire_idx((j + 2) % 4, ci + 2)
            else:
                @pl.when(r <= my_rounds - 2)
                def _():
                    fire_idx((j + 2) % 4, ci + 2)
        return 0
    lax.fori_loop(0, my_rounds, round_body, 0)
    wait_scatter(1, 3)
    plsc.subcore_barrier()

    @pl.when(c == 0)
    def _():
        pltpu.sync_copy(aggr_sh.at[pl.ds(s * ROWS_PER_TILE, ROWS_PER_TILE)],
                        out0_hbm.at[pl.ds(s * ROWS_PER_TILE, ROWS_PER_TILE)])

    @pl.when(c == 1)
    def _():
        pltpu.sync_copy(aggr_sh.at[pl.ds(s * ROWS_PER_TILE, ROWS_PER_TILE)],
                        out1_hbm.at[pl.ds(s * ROWS_PER_TILE, ROWS_PER_TILE)])


@functools.cache
def _get_sc_edge():
  return pl.kernel(
    _sc_edge_body,
    out_type=(
        jax.ShapeDtypeStruct((N_PAD, HIDDEN), jnp.float32),
        jax.ShapeDtypeStruct((N_PAD, HIDDEN), jnp.float32),
    ),
    mesh=plsc.VectorSubcoreMesh(core_axis_name="c", subcore_axis_name="s",
                                num_cores=NC, num_subcores=NS),
    scratch_types=[
        pltpu.VMEM((4, CHUNK), jnp.int32),
        pltpu.VMEM((4, CHUNK), jnp.int32),
        pltpu.VMEM((4, CHUNK), jnp.int32),
        pltpu.VMEM((2, CHUNK, HIDDEN), jnp.float32),
        pltpu.VMEM((2, CHUNK, HIDDEN), jnp.float32),
        pltpu.SemaphoreType.DMA,
        pltpu.SemaphoreType.DMA,
        pltpu.SemaphoreType.DMA,
        pltpu.VMEM_SHARED((N_PAD, HIDDEN), jnp.float32),
    ],
  )


def _mlp_body(h_ref, a0_ref, a1_ref, scale_ref, w1_ref, b1_ref, w2_ref, b2_ref,
              out_ref):
    z = h_ref[...] * scale_ref[0, 0] + a0_ref[...] + a1_ref[...]
    t = jnp.dot(z, w1_ref[...], preferred_element_type=jnp.float32) + b1_ref[...]
    t = jnp.maximum(t, 0.0)
    out_ref[...] = (jnp.dot(t, w2_ref[...], preferred_element_type=jnp.float32)
                    + b2_ref[...])


def _mlp(h, a0, a1, scale, w1, b1, w2, b2):
    blk = 1000
    grid = N_NODES // blk
    return pl.pallas_call(
        _mlp_body,
        grid=(grid,),
        in_specs=[
            pl.BlockSpec((blk, HIDDEN), lambda i: (i, 0)),
            pl.BlockSpec((blk, HIDDEN), lambda i: (i, 0)),
            pl.BlockSpec((blk, HIDDEN), lambda i: (i, 0)),
            pl.BlockSpec((1, 1), lambda i: (0, 0)),
            pl.BlockSpec((HIDDEN, 2 * HIDDEN), lambda i: (0, 0)),
            pl.BlockSpec((1, 2 * HIDDEN), lambda i: (0, 0)),
            pl.BlockSpec((2 * HIDDEN, HIDDEN), lambda i: (0, 0)),
            pl.BlockSpec((1, HIDDEN), lambda i: (0, 0)),
        ],
        out_specs=pl.BlockSpec((blk, HIDDEN), lambda i: (i, 0)),
        out_shape=jax.ShapeDtypeStruct((N_NODES, HIDDEN), jnp.float32),
    )(h, a0, a1, scale, w1, b1, w2, b2)


def _readout_body(h_ref, batch_ref, wc_ref, bc_ref, logits_ref, gf_ref):
    onehot = (batch_ref[...] ==
              lax.broadcasted_iota(jnp.int32, (N_NODES, NUM_GRAPHS), 1)
              ).astype(jnp.float32)
    sums = lax.dot_general(onehot, h_ref[...], (((0,), (0,)), ((), ())),
                           preferred_element_type=jnp.float32)
    counts = jnp.sum(onehot, axis=0)[:, None]
    gf = sums / jnp.maximum(counts, 1.0)
    logits_ref[...] = (jnp.dot(gf, wc_ref[...], preferred_element_type=jnp.float32)
                       + bc_ref[...])
    gf_ref[...] = gf


def _readout(h, batch2d, wc, bc):
    return pl.pallas_call(
        _readout_body,
        out_shape=(
            jax.ShapeDtypeStruct((NUM_GRAPHS, OUT_DIM), jnp.float32),
            jax.ShapeDtypeStruct((NUM_GRAPHS, HIDDEN), jnp.float32),
        ),
    )(h, batch2d, wc, bc)


def kernel(x, edge_index, edge_attr, batch, atom_emb, bond_emb, eps,
           W1, b1, W2, b2, Wc, bc):
    src = edge_index[0]
    dst = edge_index[1]
    eidx = (edge_attr[:, 0] + BOND_VOCAB * edge_attr[:, 1]
            + BOND_VOCAB * BOND_VOCAB * edge_attr[:, 2])
    pad = E_PAD - N_EDGES
    src_p = jnp.concatenate([src, jnp.zeros((pad,), jnp.int32)])
    dst_p = jnp.concatenate([dst, jnp.full((pad,), N_NODES, jnp.int32)])
    eidx_p = jnp.concatenate([eidx, jnp.zeros((pad,), jnp.int32)])

    tables = _bond_tables(bond_emb)
    h = _atom_encoder(x, atom_emb)

    b1_2d = b1.reshape(NUM_LAYERS, 1, 2 * HIDDEN)
    b2_2d = b2.reshape(NUM_LAYERS, 1, HIDDEN)
    scales = (1.0 + eps).reshape(NUM_LAYERS, 1, 1)

    for l in range(NUM_LAYERS):
        a0, a1 = _get_sc_edge()(h, src_p, dst_p, eidx_p, tables[l])
        h = _mlp(h, a0, a1, scales[l], W1[l], b1_2d[l], W2[l], b2_2d[l])

    logits, gf = _readout(h, batch[:, None], Wc, bc)
    return (logits, gf)


# speedup vs baseline: 1.5736x; 1.3564x over previous
import functools

import jax
import jax.numpy as jnp
import numpy as np
from jax import lax
from jax.experimental import pallas as pl
from jax.experimental.pallas import tpu as pltpu
from jax.experimental.pallas import tpu_sc as plsc

N_NODES = 10000
N_EDGES = 320000
HIDDEN = 128
NUM_LAYERS = 3
NUM_GRAPHS = 128
OUT_DIM = 10
N_ATOM_FEATS = 9
N_BOND_FEATS = 3
ATOM_VOCAB = 128
BOND_VOCAB = 16

NC = 2
NS = 16
NW = NC * NS
CHUNK = 80
CPW = 126
CPW0 = 148
CPW1 = 104
E_PAD = NW * CPW * CHUNK
N_PAD = 10240
ROWS_PER_TILE = N_PAD // NS
VEC = 16

HP = HIDDEN // 2

def _half_select(offset):
    p = np.zeros((HIDDEN, HP), dtype=np.float32)
    for k in range(HIDDEN // 32):
        for i in range(16):
            p[32 * k + offset + i, 16 * k + i] = 1.0
    return p

_P_LO = _half_select(0)
_P_HI = _half_select(16)


def _pack_rows(x, plo, phi):
    a = jnp.dot(x, plo, preferred_element_type=jnp.float32)
    b = jnp.dot(x, phi, preferred_element_type=jnp.float32)
    a = a.astype(jnp.bfloat16).astype(jnp.float32)
    b = b.astype(jnp.bfloat16).astype(jnp.float32)
    ai = lax.bitcast_convert_type(a, jnp.int32)
    bi = lax.bitcast_convert_type(b, jnp.int32)
    return lax.shift_right_logical(ai, 16) | (bi & jnp.int32(-65536))


def _atom_encoder_body(x_ref, emb_ref, plo_ref, phi_ref, out_ref, outp_ref):
    xb = x_ref[...]
    blk = xb.shape[0]
    acc = jnp.zeros((blk, HIDDEN), dtype=jnp.float32)
    iota = lax.broadcasted_iota(jnp.int32, (blk, ATOM_VOCAB), 1)
    for f in range(N_ATOM_FEATS):
        onehot = (xb[:, f][:, None] == iota).astype(jnp.float32)
        acc = acc + jnp.dot(onehot, emb_ref[f],
                            preferred_element_type=jnp.float32)
    out_ref[...] = acc
    outp_ref[...] = _pack_rows(acc, plo_ref[...], phi_ref[...])


def _atom_encoder(x, atom_emb, plo, phi):
    blk = 1000
    grid = N_NODES // blk
    return pl.pallas_call(
        _atom_encoder_body,
        grid=(grid,),
        in_specs=[
            pl.BlockSpec((blk, N_ATOM_FEATS), lambda i: (i, 0)),
            pl.BlockSpec((N_ATOM_FEATS, ATOM_VOCAB, HIDDEN), lambda i: (0, 0, 0)),
            pl.BlockSpec((HIDDEN, HP), lambda i: (0, 0)),
            pl.BlockSpec((HIDDEN, HP), lambda i: (0, 0)),
        ],
        out_specs=(pl.BlockSpec((blk, HIDDEN), lambda i: (i, 0)),
                   pl.BlockSpec((blk, HP), lambda i: (i, 0))),
        out_shape=(jax.ShapeDtypeStruct((N_NODES, HIDDEN), jnp.float32),
                   jax.ShapeDtypeStruct((N_NODES, HP), jnp.int32)),
    )(x, atom_emb, plo, phi)


def _bond_tables_body(bond_ref, plo_ref, phi_ref, out_ref):
    for l in range(NUM_LAYERS):
        t0 = bond_ref[l, 0]
        t1 = bond_ref[l, 1]
        t2 = bond_ref[l, 2]
        t = (t2[:, None, None, :] + t1[None, :, None, :] + t0[None, None, :, :])
        t = t.reshape(BOND_VOCAB ** 3, HIDDEN)
        out_ref[l] = _pack_rows(t, plo_ref[...], phi_ref[...])


def _bond_tables(bond_emb, plo, phi):
    return pl.pallas_call(
        _bond_tables_body,
        out_shape=jax.ShapeDtypeStruct(
            (NUM_LAYERS, BOND_VOCAB ** 3, HP), jnp.int32),
    )(bond_emb, plo, phi)


def _sc_edge_body(h_hbm, src_hbm, dst_hbm, eidx_hbm, table_hbm,
                  out0_hbm, out1_hbm,
                  src_v, dst_v, eidx_v, hrows_v, erows_v, msg_v,
                  sem_i, sem_g, sem_s,
                  aggr_sh):
    c = lax.axis_index("c")
    s = lax.axis_index("s")
    chunk0 = lax.select(c == 0, s * CPW0, NS * CPW0 + s * CPW1)
    my_rounds = lax.select(c == 0, CPW0 // 4, CPW1 // 4)

    @plsc.parallel_loop(0, CHUNK, 1, unroll=4)
    def _zfill(j):
        for k in range(HIDDEN // VEC):
            msg_v[0, j, pl.ds(k * VEC, VEC)] = jnp.zeros((VEC,), jnp.float32)
    for r in range(ROWS_PER_TILE // CHUNK):
        pltpu.sync_copy(
            msg_v.at[0], aggr_sh.at[pl.ds(s * ROWS_PER_TILE + r * CHUNK, CHUNK)])
    plsc.subcore_barrier()

    def fire_idx(q, ci):
        base = (chunk0 + ci) * CHUNK
        pltpu.async_copy(src_hbm.at[pl.ds(base, CHUNK)], src_v.at[q], sem_i)
        pltpu.async_copy(eidx_hbm.at[pl.ds(base, CHUNK)], eidx_v.at[q], sem_i)
        pltpu.async_copy(dst_hbm.at[pl.ds(base, CHUNK)], dst_v.at[q], sem_i)

    def wait_idx(q, ci):
        base = (chunk0 + ci) * CHUNK
        pltpu.make_async_copy(src_hbm.at[pl.ds(base, CHUNK)], src_v.at[q], sem_i).wait()
        pltpu.make_async_copy(eidx_hbm.at[pl.ds(base, CHUNK)], eidx_v.at[q], sem_i).wait()
        pltpu.make_async_copy(dst_hbm.at[pl.ds(base, CHUNK)], dst_v.at[q], sem_i).wait()

    def fire_gather(b, q):
        pltpu.async_copy(h_hbm.at[src_v.at[q]], hrows_v.at[b], sem_g)
        pltpu.async_copy(table_hbm.at[eidx_v.at[q]], erows_v.at[b], sem_g)

    def wait_gather(b, q):
        pltpu.make_async_copy(h_hbm.at[src_v.at[q]], hrows_v.at[b], sem_g).wait()
        pltpu.make_async_copy(table_hbm.at[eidx_v.at[q]], erows_v.at[b], sem_g).wait()

    def fire_scatter(b, q):
        pltpu.async_copy(msg_v.at[b], aggr_sh.at[dst_v.at[q]], sem_s, add=True)

    def wait_scatter(b, q):
        pltpu.make_async_copy(msg_v.at[b], aggr_sh.at[dst_v.at[q]], sem_s).wait()

    _mask = jnp.full((VEC,), -65536, jnp.int32)

    def compute(b):
        @plsc.parallel_loop(0, CHUNK, 1, unroll=2)
        def _(j):
            for k in range(HP // VEC):
                sl = pl.ds(k * VEC, VEC)
                hb = hrows_v[b, j, sl]
                eb = erows_v[b, j, sl]
                h_lo = lax.bitcast_convert_type(jnp.left_shift(hb, 16), jnp.float32)
                e_lo = lax.bitcast_convert_type(jnp.left_shift(eb, 16), jnp.float32)
                h_hi = lax.bitcast_convert_type(hb & _mask, jnp.float32)
                e_hi = lax.bitcast_convert_type(eb & _mask, jnp.float32)
                msg_v[b, j, pl.ds(2 * k * VEC, VEC)] = (
                    jnp.maximum(h_lo + e_lo, 0.0))
                msg_v[b, j, pl.ds((2 * k + 1) * VEC, VEC)] = (
                    jnp.maximum(h_hi + e_hi, 0.0))

    fire_idx(0, 0)
    wait_idx(0, 0)
    fire_gather(0, 0)
    fire_idx(1, 1)

    def round_body(r, _):
        for j in range(4):
            ci = 4 * r + j
            b, q = j % 2, j
            bn, qn = (j + 1) % 2, (j + 1) % 4
            bp, qp = (j - 1) % 2, (j - 1) % 4

            if j == 0:
                @pl.when(r >= 1)
                def _():
                    wait_scatter(bp, qp)
            else:
                wait_scatter(bp, qp)

            if j <= 2:
                wait_idx(qn, ci + 1)
                fire_gather(bn, qn)
            else:
                @pl.when(r <= my_rounds - 2)
                def _():
                    wait_idx(qn, ci + 1)
                    fire_gather(bn, qn)

            wait_gather(b, q)
            compute(b)
            fire_scatter(b, q)

            if j <= 1:
                fire_idx((j + 2) % 4, ci + 2)
            else:
                @pl.when(r <= my_rounds - 2)
                def _():
                    fire_idx((j + 2) % 4, ci + 2)
        return 0
    lax.fori_loop(0, my_rounds, round_body, 0)
    wait_scatter(1, 3)
    plsc.subcore_barrier()

    @pl.when(c == 0)
    def _():
        pltpu.sync_copy(aggr_sh.at[pl.ds(s * ROWS_PER_TILE, ROWS_PER_TILE)],
                        out0_hbm.at[pl.ds(s * ROWS_PER_TILE, ROWS_PER_TILE)])

    @pl.when(c == 1)
    def _():
        pltpu.sync_copy(aggr_sh.at[pl.ds(s * ROWS_PER_TILE, ROWS_PER_TILE)],
                        out1_hbm.at[pl.ds(s * ROWS_PER_TILE, ROWS_PER_TILE)])


@functools.cache
def _get_sc_edge():
  return pl.kernel(
    _sc_edge_body,
    out_type=(
        jax.ShapeDtypeStruct((N_PAD, HIDDEN), jnp.float32),
        jax.ShapeDtypeStruct((N_PAD, HIDDEN), jnp.float32),
    ),
    mesh=plsc.VectorSubcoreMesh(core_axis_name="c", subcore_axis_name="s",
                                num_cores=NC, num_subcores=NS),
    compiler_params=pltpu.CompilerParams(use_tc_tiling_on_sc=False),
    scratch_types=[
        pltpu.VMEM((4, CHUNK), jnp.int32),
        pltpu.VMEM((4, CHUNK), jnp.int32),
        pltpu.VMEM((4, CHUNK), jnp.int32),
        pltpu.VMEM((2, CHUNK, HP), jnp.int32),
        pltpu.VMEM((2, CHUNK, HP), jnp.int32),
        pltpu.VMEM((2, CHUNK, HIDDEN), jnp.float32),
        pltpu.SemaphoreType.DMA,
        pltpu.SemaphoreType.DMA,
        pltpu.SemaphoreType.DMA,
        pltpu.VMEM_SHARED((N_PAD, HIDDEN), jnp.float32),
    ],
  )


def _mlp_body(h_ref, a0_ref, a1_ref, scale_ref, w1_ref, b1_ref, w2_ref, b2_ref,
              plo_ref, phi_ref, out_ref, outp_ref):
    z = h_ref[...] * scale_ref[0, 0] + a0_ref[...] + a1_ref[...]
    t = jnp.dot(z, w1_ref[...], preferred_element_type=jnp.float32) + b1_ref[...]
    t = jnp.maximum(t, 0.0)
    h_new = (jnp.dot(t, w2_ref[...], preferred_element_type=jnp.float32)
             + b2_ref[...])
    out_ref[...] = h_new
    outp_ref[...] = _pack_rows(h_new, plo_ref[...], phi_ref[...])


def _mlp(h, a0, a1, scale, w1, b1, w2, b2, plo, phi):
    blk = 1000
    grid = N_NODES // blk
    return pl.pallas_call(
        _mlp_body,
        grid=(grid,),
        in_specs=[
            pl.BlockSpec((blk, HIDDEN), lambda i: (i, 0)),
            pl.BlockSpec((blk, HIDDEN), lambda i: (i, 0)),
            pl.BlockSpec((blk, HIDDEN), lambda i: (i, 0)),
            pl.BlockSpec((1, 1), lambda i: (0, 0)),
            pl.BlockSpec((HIDDEN, 2 * HIDDEN), lambda i: (0, 0)),
            pl.BlockSpec((1, 2 * HIDDEN), lambda i: (0, 0)),
            pl.BlockSpec((2 * HIDDEN, HIDDEN), lambda i: (0, 0)),
            pl.BlockSpec((1, HIDDEN), lambda i: (0, 0)),
            pl.BlockSpec((HIDDEN, HP), lambda i: (0, 0)),
            pl.BlockSpec((HIDDEN, HP), lambda i: (0, 0)),
        ],
        out_specs=(pl.BlockSpec((blk, HIDDEN), lambda i: (i, 0)),
                   pl.BlockSpec((blk, HP), lambda i: (i, 0))),
        out_shape=(jax.ShapeDtypeStruct((N_NODES, HIDDEN), jnp.float32),
                   jax.ShapeDtypeStruct((N_NODES, HP), jnp.int32)),
    )(h, a0, a1, scale, w1, b1, w2, b2, plo, phi)


def _readout_body(h_ref, batch_ref, wc_ref, bc_ref, logits_ref, gf_ref):
    onehot = (batch_ref[...] ==
              lax.broadcasted_iota(jnp.int32, (N_NODES, NUM_GRAPHS), 1)
              ).astype(jnp.float32)
    sums = lax.dot_general(onehot, h_ref[...], (((0,), (0,)), ((), ())),
                           preferred_element_type=jnp.float32)
    counts = jnp.sum(onehot, axis=0)[:, None]
    gf = sums / jnp.maximum(counts, 1.0)
    logits_ref[...] = (jnp.dot(gf, wc_ref[...], preferred_element_type=jnp.float32)
                       + bc_ref[...])
    gf_ref[...] = gf


def _readout(h, batch2d, wc, bc):
    return pl.pallas_call(
        _readout_body,
        out_shape=(
            jax.ShapeDtypeStruct((NUM_GRAPHS, OUT_DIM), jnp.float32),
            jax.ShapeDtypeStruct((NUM_GRAPHS, HIDDEN), jnp.float32),
        ),
    )(h, batch2d, wc, bc)


def kernel(x, edge_index, edge_attr, batch, atom_emb, bond_emb, eps,
           W1, b1, W2, b2, Wc, bc):
    src = edge_index[0]
    dst = edge_index[1]
    eidx = (edge_attr[:, 0] + BOND_VOCAB * edge_attr[:, 1]
            + BOND_VOCAB * BOND_VOCAB * edge_attr[:, 2])
    pad = E_PAD - N_EDGES
    src_p = jnp.concatenate([src, jnp.zeros((pad,), jnp.int32)])
    dst_p = jnp.concatenate([dst, jnp.full((pad,), N_NODES, jnp.int32)])
    eidx_p = jnp.concatenate([eidx, jnp.zeros((pad,), jnp.int32)])

    plo = jnp.asarray(_P_LO)
    phi = jnp.asarray(_P_HI)
    tables = _bond_tables(bond_emb, plo, phi)
    h, h_pk = _atom_encoder(x, atom_emb, plo, phi)

    b1_2d = b1.reshape(NUM_LAYERS, 1, 2 * HIDDEN)
    b2_2d = b2.reshape(NUM_LAYERS, 1, HIDDEN)
    scales = (1.0 + eps).reshape(NUM_LAYERS, 1, 1)

    for l in range(NUM_LAYERS):
        a0, a1 = _get_sc_edge()(h_pk, src_p, dst_p, eidx_p, tables[l])
        h, h_pk = _mlp(h, a0, a1, scales[l], W1[l], b1_2d[l], W2[l], b2_2d[l],
                       plo, phi)

    logits, gf = _readout(h, batch[:, None], Wc, bc)
    return (logits, gf)


# speedup vs baseline: 1.6266x; 1.0337x over previous
import functools

import jax
import jax.numpy as jnp
import numpy as np
from jax import lax
from jax.experimental import pallas as pl
from jax.experimental.pallas import tpu as pltpu
from jax.experimental.pallas import tpu_sc as plsc

N_NODES = 10000
N_EDGES = 320000
HIDDEN = 128
NUM_LAYERS = 3
NUM_GRAPHS = 128
OUT_DIM = 10
N_ATOM_FEATS = 9
N_BOND_FEATS = 3
ATOM_VOCAB = 128
BOND_VOCAB = 16

NC = 2
NS = 16
NW = NC * NS
CHUNK = 80
CPW = 126
CPW0 = 148
CPW1 = 104
E_PAD = NW * CPW * CHUNK
N_PAD = 10240
ROWS_PER_TILE = N_PAD // NS
VEC = 16

HP = HIDDEN // 2

def _half_select(offset):
    p = np.zeros((HIDDEN, HP), dtype=np.float32)
    for k in range(HIDDEN // 32):
        for i in range(16):
            p[32 * k + offset + i, 16 * k + i] = 1.0
    return p

_P_LO = _half_select(0)
_P_HI = _half_select(16)


def _pack_rows(x, plo, phi):
    a = jnp.dot(x, plo, preferred_element_type=jnp.float32)
    b = jnp.dot(x, phi, preferred_element_type=jnp.float32)
    a = a.astype(jnp.bfloat16).astype(jnp.float32)
    b = b.astype(jnp.bfloat16).astype(jnp.float32)
    ai = lax.bitcast_convert_type(a, jnp.int32)
    bi = lax.bitcast_convert_type(b, jnp.int32)
    return lax.shift_right_logical(ai, 16) | (bi & jnp.int32(-65536))


def _atom_encoder_body(x_ref, emb_ref, bond_ref, plo_ref, phi_ref,
                       out_ref, outp_ref, tab_ref):
    xb = x_ref[...]
    blk = xb.shape[0]
    acc = jnp.zeros((blk, HIDDEN), dtype=jnp.float32)
    iota = lax.broadcasted_iota(jnp.int32, (blk, ATOM_VOCAB), 1)
    for f in range(N_ATOM_FEATS):
        onehot = (xb[:, f][:, None] == iota).astype(jnp.float32)
        acc = acc + jnp.dot(onehot, emb_ref[f],
                            preferred_element_type=jnp.float32)
    out_ref[...] = acc
    outp_ref[...] = _pack_rows(acc, plo_ref[...], phi_ref[...])

    @pl.when(pl.program_id(0) == 0)
    def _():
        for l in range(NUM_LAYERS):
            t0 = bond_ref[l, 0]
            t1 = bond_ref[l, 1]
            t2 = bond_ref[l, 2]
            t = (t2[:, None, None, :] + t1[None, :, None, :]
                 + t0[None, None, :, :])
            t = t.reshape(BOND_VOCAB ** 3, HIDDEN)
            tab_ref[l] = _pack_rows(t, plo_ref[...], phi_ref[...])


def _atom_encoder(x, atom_emb, bond_emb, plo, phi):
    blk = 1000
    grid = N_NODES // blk
    return pl.pallas_call(
        _atom_encoder_body,
        grid=(grid,),
        in_specs=[
            pl.BlockSpec((blk, N_ATOM_FEATS), lambda i: (i, 0)),
            pl.BlockSpec((N_ATOM_FEATS, ATOM_VOCAB, HIDDEN), lambda i: (0, 0, 0)),
            pl.BlockSpec(
                (NUM_LAYERS, N_BOND_FEATS, BOND_VOCAB, HIDDEN),
                lambda i: (0, 0, 0, 0)),
            pl.BlockSpec((HIDDEN, HP), lambda i: (0, 0)),
            pl.BlockSpec((HIDDEN, HP), lambda i: (0, 0)),
        ],
        out_specs=(pl.BlockSpec((blk, HIDDEN), lambda i: (i, 0)),
                   pl.BlockSpec((blk, HP), lambda i: (i, 0)),
                   pl.BlockSpec((NUM_LAYERS, BOND_VOCAB ** 3, HP),
                                lambda i: (0, 0, 0))),
        out_shape=(jax.ShapeDtypeStruct((N_NODES, HIDDEN), jnp.float32),
                   jax.ShapeDtypeStruct((N_NODES, HP), jnp.int32),
                   jax.ShapeDtypeStruct((NUM_LAYERS, BOND_VOCAB ** 3, HP),
                                        jnp.int32)),
    )(x, atom_emb, bond_emb, plo, phi)


def _sc_edge_body(h_hbm, src_hbm, dst_hbm, eidx_hbm, table_hbm,
                  out0_hbm, out1_hbm,
                  src_v, dst_v, eidx_v, hrows_v, erows_v, msg_v,
                  sem_i, sem_g, sem_s,
                  aggr_sh):
    c = lax.axis_index("c")
    s = lax.axis_index("s")
    chunk0 = lax.select(c == 0, s * CPW0, NS * CPW0 + s * CPW1)
    my_rounds = lax.select(c == 0, CPW0 // 4, CPW1 // 4)

    @plsc.parallel_loop(0, CHUNK, 1, unroll=4)
    def _zfill(j):
        for k in range(HIDDEN // VEC):
            msg_v[0, j, pl.ds(k * VEC, VEC)] = jnp.zeros((VEC,), jnp.float32)
    for r in range(ROWS_PER_TILE // CHUNK):
        pltpu.sync_copy(
            msg_v.at[0], aggr_sh.at[pl.ds(s * ROWS_PER_TILE + r * CHUNK, CHUNK)])
    plsc.subcore_barrier()

    def fire_idx(q, ci):
        base = (chunk0 + ci) * CHUNK
        pltpu.async_copy(src_hbm.at[pl.ds(base, CHUNK)], src_v.at[q], sem_i)
        pltpu.async_copy(eidx_hbm.at[pl.ds(base, CHUNK)], eidx_v.at[q], sem_i)
        pltpu.async_copy(dst_hbm.at[pl.ds(base, CHUNK)], dst_v.at[q], sem_i)

    def wait_idx(q, ci):
        base = (chunk0 + ci) * CHUNK
        pltpu.make_async_copy(src_hbm.at[pl.ds(base, CHUNK)], src_v.at[q], sem_i).wait()
        pltpu.make_async_copy(eidx_hbm.at[pl.ds(base, CHUNK)], eidx_v.at[q], sem_i).wait()
        pltpu.make_async_copy(dst_hbm.at[pl.ds(base, CHUNK)], dst_v.at[q], sem_i).wait()

    def fire_gather(b, q):
        pltpu.async_copy(h_hbm.at[src_v.at[q]], hrows_v.at[b], sem_g)
        pltpu.async_copy(table_hbm.at[eidx_v.at[q]], erows_v.at[b], sem_g)

    def wait_gather(b, q):
        pltpu.make_async_copy(h_hbm.at[src_v.at[q]], hrows_v.at[b], sem_g).wait()
        pltpu.make_async_copy(table_hbm.at[eidx_v.at[q]], erows_v.at[b], sem_g).wait()

    def fire_scatter(b, q):
        pltpu.async_copy(msg_v.at[b], aggr_sh.at[dst_v.at[q]], sem_s, add=True)

    def wait_scatter(b, q):
        pltpu.make_async_copy(msg_v.at[b], aggr_sh.at[dst_v.at[q]], sem_s).wait()

    _mask = jnp.full((VEC,), -65536, jnp.int32)

    def compute(b):
        @plsc.parallel_loop(0, CHUNK, 1, unroll=2)
        def _(j):
            for k in range(HP // VEC):
                sl = pl.ds(k * VEC, VEC)
                hb = hrows_v[b, j, sl]
                eb = erows_v[b, j, sl]
                h_lo = lax.bitcast_convert_type(jnp.left_shift(hb, 16), jnp.float32)
                e_lo = lax.bitcast_convert_type(jnp.left_shift(eb, 16), jnp.float32)
                h_hi = lax.bitcast_convert_type(hb & _mask, jnp.float32)
                e_hi = lax.bitcast_convert_type(eb & _mask, jnp.float32)
                msg_v[b, j, pl.ds(2 * k * VEC, VEC)] = (
                    jnp.maximum(h_lo + e_lo, 0.0))
                msg_v[b, j, pl.ds((2 * k + 1) * VEC, VEC)] = (
                    jnp.maximum(h_hi + e_hi, 0.0))

    fire_idx(0, 0)
    wait_idx(0, 0)
    fire_gather(0, 0)
    fire_idx(1, 1)

    def round_body(r, _):
        for j in range(4):
            ci = 4 * r + j
            b, q = j % 2, j
            bn, qn = (j + 1) % 2, (j + 1) % 4
            bp, qp = (j - 1) % 2, (j - 1) % 4

            if j == 0:
                @pl.when(r >= 1)
                def _():
                    wait_scatter(bp, qp)
            else:
                wait_scatter(bp, qp)

            if j <= 2:
                wait_idx(qn, ci + 1)
                fire_gather(bn, qn)
            else:
                @pl.when(r <= my_rounds - 2)
                def _():
                    wait_idx(qn, ci + 1)
                    fire_gather(bn, qn)

            wait_gather(b, q)
            compute(b)
            fire_scatter(b, q)

            if j <= 1:
                fire_idx((j + 2) % 4, ci + 2)
            else:
                @pl.when(r <= my_rounds - 2)
                def _():
                    fire_idx((j + 2) % 4, ci + 2)
        return 0
    lax.fori_loop(0, my_rounds, round_body, 0)
    wait_scatter(1, 3)
    plsc.subcore_barrier()

    @pl.when(c == 0)
    def _():
        pltpu.sync_copy(aggr_sh.at[pl.ds(s * ROWS_PER_TILE, ROWS_PER_TILE)],
                        out0_hbm.at[pl.ds(s * ROWS_PER_TILE, ROWS_PER_TILE)])

    @pl.when(c == 1)
    def _():
        pltpu.sync_copy(aggr_sh.at[pl.ds(s * ROWS_PER_TILE, ROWS_PER_TILE)],
                        out1_hbm.at[pl.ds(s * ROWS_PER_TILE, ROWS_PER_TILE)])


@functools.cache
def _get_sc_edge():
  return pl.kernel(
    _sc_edge_body,
    out_type=(
        jax.ShapeDtypeStruct((N_PAD, HIDDEN), jnp.float32),
        jax.ShapeDtypeStruct((N_PAD, HIDDEN), jnp.float32),
    ),
    mesh=plsc.VectorSubcoreMesh(core_axis_name="c", subcore_axis_name="s",
                                num_cores=NC, num_subcores=NS),
    compiler_params=pltpu.CompilerParams(use_tc_tiling_on_sc=False),
    scratch_types=[
        pltpu.VMEM((4, CHUNK), jnp.int32),
        pltpu.VMEM((4, CHUNK), jnp.int32),
        pltpu.VMEM((4, CHUNK), jnp.int32),
        pltpu.VMEM((2, CHUNK, HP), jnp.int32),
        pltpu.VMEM((2, CHUNK, HP), jnp.int32),
        pltpu.VMEM((2, CHUNK, HIDDEN), jnp.float32),
        pltpu.SemaphoreType.DMA,
        pltpu.SemaphoreType.DMA,
        pltpu.SemaphoreType.DMA,
        pltpu.VMEM_SHARED((N_PAD, HIDDEN), jnp.float32),
    ],
  )


def _mlp_body(h_ref, a0_ref, a1_ref, scale_ref, w1_ref, b1_ref, w2_ref, b2_ref,
              plo_ref, phi_ref, out_ref, outp_ref):
    z = h_ref[...] * scale_ref[0, 0] + a0_ref[...] + a1_ref[...]
    t = jnp.dot(z, w1_ref[...], preferred_element_type=jnp.float32) + b1_ref[...]
    t = jnp.maximum(t, 0.0)
    h_new = (jnp.dot(t, w2_ref[...], preferred_element_type=jnp.float32)
             + b2_ref[...])
    out_ref[...] = h_new
    outp_ref[...] = _pack_rows(h_new, plo_ref[...], phi_ref[...])


def _mlp(h, a0, a1, scale, w1, b1, w2, b2, plo, phi):
    blk = 1000
    grid = N_NODES // blk
    return pl.pallas_call(
        _mlp_body,
        grid=(grid,),
        in_specs=[
            pl.BlockSpec((blk, HIDDEN), lambda i: (i, 0)),
            pl.BlockSpec((blk, HIDDEN), lambda i: (i, 0)),
            pl.BlockSpec((blk, HIDDEN), lambda i: (i, 0)),
            pl.BlockSpec((1, 1), lambda i: (0, 0)),
            pl.BlockSpec((HIDDEN, 2 * HIDDEN), lambda i: (0, 0)),
            pl.BlockSpec((1, 2 * HIDDEN), lambda i: (0, 0)),
            pl.BlockSpec((2 * HIDDEN, HIDDEN), lambda i: (0, 0)),
            pl.BlockSpec((1, HIDDEN), lambda i: (0, 0)),
            pl.BlockSpec((HIDDEN, HP), lambda i: (0, 0)),
            pl.BlockSpec((HIDDEN, HP), lambda i: (0, 0)),
        ],
        out_specs=(pl.BlockSpec((blk, HIDDEN), lambda i: (i, 0)),
                   pl.BlockSpec((blk, HP), lambda i: (i, 0))),
        out_shape=(jax.ShapeDtypeStruct((N_NODES, HIDDEN), jnp.float32),
                   jax.ShapeDtypeStruct((N_NODES, HP), jnp.int32)),
    )(h, a0, a1, scale, w1, b1, w2, b2, plo, phi)


def _mlp_readout_body(h_ref, a0_ref, a1_ref, scale_ref, w1_ref, b1_ref,
                      w2_ref, b2_ref, batch_ref, wc_ref, bc_ref,
                      logits_ref, gf_ref, sums_ref, counts_ref):
    i = pl.program_id(0)
    n = pl.num_programs(0)
    z = h_ref[...] * scale_ref[0, 0] + a0_ref[...] + a1_ref[...]
    t = jnp.dot(z, w1_ref[...], preferred_element_type=jnp.float32) + b1_ref[...]
    t = jnp.maximum(t, 0.0)
    h_new = (jnp.dot(t, w2_ref[...], preferred_element_type=jnp.float32)
             + b2_ref[...])
    blk = h_new.shape[0]
    onehot = (batch_ref[...] ==
              lax.broadcasted_iota(jnp.int32, (blk, NUM_GRAPHS), 1)
              ).astype(jnp.float32)
    psum = lax.dot_general(onehot, h_new, (((0,), (0,)), ((), ())),
                           preferred_element_type=jnp.float32)
    pcnt = jnp.sum(onehot, axis=0)[:, None]

    @pl.when(i == 0)
    def _():
        sums_ref[...] = jnp.zeros_like(sums_ref)
        counts_ref[...] = jnp.zeros_like(counts_ref)

    sums_ref[...] += psum
    counts_ref[...] += pcnt

    @pl.when(i == n - 1)
    def _():
        gf = sums_ref[...] / jnp.maximum(counts_ref[...], 1.0)
        logits_ref[...] = (jnp.dot(gf, wc_ref[...],
                                   preferred_element_type=jnp.float32)
                           + bc_ref[...])
        gf_ref[...] = gf


def _mlp_readout(h, a0, a1, scale, w1, b1, w2, b2, batch2d, wc, bc):
    blk = 1000
    grid = N_NODES // blk
    return pl.pallas_call(
        _mlp_readout_body,
        grid=(grid,),
        in_specs=[
            pl.BlockSpec((blk, HIDDEN), lambda i: (i, 0)),
            pl.BlockSpec((blk, HIDDEN), lambda i: (i, 0)),
            pl.BlockSpec((blk, HIDDEN), lambda i: (i, 0)),
            pl.BlockSpec((1, 1), lambda i: (0, 0)),
            pl.BlockSpec((HIDDEN, 2 * HIDDEN), lambda i: (0, 0)),
            pl.BlockSpec((1, 2 * HIDDEN), lambda i: (0, 0)),
            pl.BlockSpec((2 * HIDDEN, HIDDEN), lambda i: (0, 0)),
            pl.BlockSpec((1, HIDDEN), lambda i: (0, 0)),
            pl.BlockSpec((blk, 1), lambda i: (i, 0)),
            pl.BlockSpec((HIDDEN, OUT_DIM), lambda i: (0, 0)),
            pl.BlockSpec((1, OUT_DIM), lambda i: (0, 0)),
        ],
        out_specs=(pl.BlockSpec((NUM_GRAPHS, OUT_DIM), lambda i: (0, 0)),
                   pl.BlockSpec((NUM_GRAPHS, HIDDEN), lambda i: (0, 0))),
        out_shape=(jax.ShapeDtypeStruct((NUM_GRAPHS, OUT_DIM), jnp.float32),
                   jax.ShapeDtypeStruct((NUM_GRAPHS, HIDDEN), jnp.float32)),
        scratch_shapes=[pltpu.VMEM((NUM_GRAPHS, HIDDEN), jnp.float32),
                        pltpu.VMEM((NUM_GRAPHS, 1), jnp.float32)],
    )(h, a0, a1, scale, w1, b1, w2, b2, batch2d, wc, bc)


def kernel(x, edge_index, edge_attr, batch, atom_emb, bond_emb, eps,
           W1, b1, W2, b2, Wc, bc):
    src = edge_index[0]
    dst = edge_index[1]
    eidx = (edge_attr[:, 0] + BOND_VOCAB * edge_attr[:, 1]
            + BOND_VOCAB * BOND_VOCAB * edge_attr[:, 2])
    pad = E_PAD - N_EDGES
    src_p = jnp.concatenate([src, jnp.zeros((pad,), jnp.int32)])
    dst_p = jnp.concatenate([dst, jnp.full((pad,), N_NODES, jnp.int32)])
    eidx_p = jnp.concatenate([eidx, jnp.zeros((pad,), jnp.int32)])

    plo = jnp.asarray(_P_LO)
    phi = jnp.asarray(_P_HI)
    h, h_pk, tables = _atom_encoder(x, atom_emb, bond_emb, plo, phi)

    b1_2d = b1.reshape(NUM_LAYERS, 1, 2 * HIDDEN)
    b2_2d = b2.reshape(NUM_LAYERS, 1, HIDDEN)
    scales = (1.0 + eps).reshape(NUM_LAYERS, 1, 1)

    for l in range(NUM_LAYERS - 1):
        a0, a1 = _get_sc_edge()(h_pk, src_p, dst_p, eidx_p, tables[l])
        h, h_pk = _mlp(h, a0, a1, scales[l], W1[l], b1_2d[l], W2[l], b2_2d[l],
                       plo, phi)

    l = NUM_LAYERS - 1
    a0, a1 = _get_sc_edge()(h_pk, src_p, dst_p, eidx_p, tables[l])
    logits, gf = _mlp_readout(h, a0, a1, scales[l], W1[l], b1_2d[l],
                              W2[l], b2_2d[l], batch[:, None],
                              Wc, bc.reshape(1, OUT_DIM))
    return (logits, gf)
